# baseline (speedup 1.0000x reference)
import functools

import jax
import jax.numpy as jnp
from jax import lax
from jax.experimental import pallas as pl
from jax.experimental.pallas import tpu as pltpu
from jax.experimental.pallas import tpu_sc as plsc

N = 10000
E = 320000
D = 128
NC = 2
NS = 16
NW = NC * NS
NPAD = 10240
NPB = NPAD // 128
ROWS_PER_TILE = NPB // NS
EB = 2500
EBP = 2560
BPT = EBP // NW
NBUF = 2
EB2 = 1250
EBP2 = 1280
BPT2 = EBP2 // NW
DEG_CHUNK = 2000

_mesh = plsc.VectorSubcoreMesh(core_axis_name="c", subcore_axis_name="s",
                               num_cores=NC, num_subcores=NS)


def _deg_body(src_hbm, dst_hbm, osrc_hbm, odst_hbm, idxv, hist, vbuf, res,
              shared):
    _Z16F = jnp.zeros((16,), jnp.float32)
    _O16F = jnp.ones((16,), jnp.float32)
    c = lax.axis_index("c")
    s = lax.axis_index("s")

    def _zero_hist(r, _):
        hist[pl.ds(r * 16, 16)] = _Z16F
        return 0
    lax.fori_loop(0, NPAD // 16, _zero_hist, 0)

    per_tile = E // NS

    def _accum():
        def _inner(j, _):
            iv = idxv[pl.ds(j * 16, 16)]
            plsc.addupdate_scatter(hist, [iv], _O16F)
            return 0
        lax.fori_loop(0, DEG_CHUNK // 16, _inner, 0)

    def _chunk(k, _):
        base = s * per_tile + k * DEG_CHUNK

        @pl.when(c == 0)
        def _():
            pltpu.sync_copy(src_hbm.at[pl.ds(base, DEG_CHUNK)], idxv)

        @pl.when(c == 1)
        def _():
            pltpu.sync_copy(dst_hbm.at[pl.ds(base, DEG_CHUNK)], idxv)

        _accum()
        return 0

    lax.fori_loop(0, per_tile // DEG_CHUNK, _chunk, 0)

    pltpu.sync_copy(hist, shared.at[pl.ds(s * NPAD, NPAD)])
    plsc.subcore_barrier()
    seg = NPAD // NS
    for k in range(NS):
        pltpu.sync_copy(shared.at[pl.ds(k * NPAD + s * seg, seg)],
                        vbuf.at[pl.ds(k * seg, seg)])

    def _red(i, _):
        a = vbuf[pl.ds(i * 16, 16)]
        for k in range(1, NS):
            a = a + vbuf[pl.ds(k * seg + i * 16, 16)]
        res[pl.ds(i * 16, 16)] = a
        return 0
    lax.fori_loop(0, seg // 16, _red, 0)

    @pl.when(c == 0)
    def _():
        pltpu.sync_copy(res, osrc_hbm.at[pl.ds(s * seg, seg)])

    @pl.when(c == 1)
    def _():
        pltpu.sync_copy(res, odst_hbm.at[pl.ds(s * seg, seg)])


_deg_call = pl.kernel(
    _deg_body,
    out_type=(jax.ShapeDtypeStruct((NPAD,), jnp.float32),
              jax.ShapeDtypeStruct((NPAD,), jnp.float32)),
    mesh=_mesh,
    compiler_params=pltpu.CompilerParams(needs_layout_passes=False),
    scratch_types=[
        pltpu.VMEM((DEG_CHUNK,), jnp.int32),
        pltpu.VMEM((NPAD,), jnp.float32),
        pltpu.VMEM((NPAD,), jnp.float32),
        pltpu.VMEM((NPAD // NS,), jnp.float32),
        pltpu.VMEM_SHARED((NS * NPAD,), jnp.float32),
    ],
)


def _gspa_body(x_hbm, src2d, dst3d, out_hbm, sidx, didx, rows, acc, sem):
    _Z16F = jnp.zeros((16,), jnp.float32)
    c = lax.axis_index("c")
    s = lax.axis_index("s")
    wid = s * NC + c

    def _zero_rows(r, _):
        for cc in range(8):
            rows[r, pl.ds(cc * 16, 16)] = _Z16F
        return 0
    lax.fori_loop(0, 128, _zero_rows, 0)
    for t in range(NPAD // NS // 128):
        pltpu.sync_copy(rows.at[pl.ds(0, 128)],
                        acc.at[pl.ds(s * (NPAD // NS) + t * 128, 128)])
    plsc.subcore_barrier()

    def _batch(j, _):
        row = wid + j * NW
        pltpu.sync_copy(src2d.at[row], sidx)
        pltpu.sync_copy(dst3d.at[row], didx)
        pltpu.async_copy(x_hbm.at[sidx], rows, sem).wait()
        pltpu.sync_copy(rows.at[pl.ds(0, 128)], acc.at[didx.at[0]], add=True)
        pltpu.sync_copy(rows.at[pl.ds(128, 128)], acc.at[didx.at[1]], add=True)
        return 0

    lax.fori_loop(0, BPT2, _batch, 0)
    plsc.subcore_barrier()

    sl = pl.ds(s * (NPAD // NS), NPAD // NS)

    @pl.when(c == 0)
    def _():
        pltpu.sync_copy(acc.at[sl], out_hbm.at[0, sl])

    @pl.when(c == 1)
    def _():
        pltpu.sync_copy(acc.at[sl], out_hbm.at[1, sl])


_gspa_call = pl.kernel(
    _gspa_body,
    out_type=jax.ShapeDtypeStruct((2, NPAD, 128), jnp.float32),
    mesh=_mesh,
    scratch_types=[
        pltpu.VMEM((256,), jnp.int32),
        pltpu.VMEM((2, 128), jnp.int32),
        pltpu.VMEM((256, 128), jnp.float32),
        pltpu.VMEM_SHARED((NPAD, 128), jnp.float32),
        pltpu.SemaphoreType.DMA,
    ],
)


def _scale_kernel(feat_ref, deg_ref, o_ref):
    norm = lax.rsqrt(jnp.maximum(deg_ref[...], 1.0))
    o_ref[...] = feat_ref[...] * norm


def _scale(feat, deg_out):
    bn = 1000
    return pl.pallas_call(
        _scale_kernel,
        out_shape=jax.ShapeDtypeStruct((N, D), jnp.float32),
        grid=(N // bn,),
        in_specs=[
            pl.BlockSpec((bn, D), lambda i: (i, 0)),
            pl.BlockSpec((bn, 1), lambda i: (i, 0)),
        ],
        out_specs=pl.BlockSpec((bn, D), lambda i: (i, 0)),
    )(feat, deg_out)


def _dense_kernel(p_ref, deg_ref, w_ref, b_ref, wih_ref, bih_ref, bhh_ref,
                  o_ref):
    norm = lax.rsqrt(jnp.maximum(deg_ref[...], 1.0))
    rst = (p_ref[0] + p_ref[1]) * norm
    h = jnp.dot(rst, w_ref[...], preferred_element_type=jnp.float32) + b_ref[...]
    gi = lax.dot_general(h, wih_ref[...], (((1,), (1,)), ((), ())),
                         preferred_element_type=jnp.float32) + bih_ref[...]
    bhh = bhh_ref[...]
    r = jax.nn.sigmoid(gi[:, 0:D] + bhh[:, 0:D])
    z = jax.nn.sigmoid(gi[:, D:2 * D] + bhh[:, D:2 * D])
    n = jnp.tanh(gi[:, 2 * D:3 * D] + r * bhh[:, 2 * D:3 * D])
    o_ref[...] = jnp.maximum((1.0 - z) * n, 0.0)


def _dense(pacc, deg_in, W, b2, w_ih, bih2, bhh2):
    bn = 400
    return pl.pallas_call(
        _dense_kernel,
        out_shape=jax.ShapeDtypeStruct((N, D), jnp.float32),
        grid=(N // bn,),
        in_specs=[
            pl.BlockSpec((2, bn, D), lambda i: (0, i, 0)),
            pl.BlockSpec((bn, 1), lambda i: (i, 0)),
            pl.BlockSpec((D, D), lambda i: (0, 0)),
            pl.BlockSpec((1, D), lambda i: (0, 0)),
            pl.BlockSpec((3 * D, D), lambda i: (0, 0)),
            pl.BlockSpec((1, 3 * D), lambda i: (0, 0)),
            pl.BlockSpec((1, 3 * D), lambda i: (0, 0)),
        ],
        out_specs=pl.BlockSpec((bn, D), lambda i: (i, 0)),
    )(pacc, deg_in, W, b2, w_ih, bih2, bhh2)


def kernel(feat, edge_index, W, b, w_ih, w_hh, b_ih, b_hh):
    src = edge_index[0]
    dst = edge_index[1]
    dsrc, ddst = _deg_call(src, dst)
    deg_out = dsrc[:N].reshape(N, 1)
    deg_in = ddst[:N].reshape(N, 1)
    x = _scale(feat, deg_out)
    pad = (EBP2 - EB2) * 256
    src_p = jnp.concatenate([src, jnp.zeros((pad,), jnp.int32)])
    dst_p = jnp.concatenate(
        [dst, N + lax.iota(jnp.int32, pad) % (NPAD - N)])
    pacc = _gspa_call(x, src_p.reshape(EBP2, 256), dst_p.reshape(EBP2, 2, 128))
    return _dense(pacc[:, :N, :], deg_in, W, b.reshape(1, D),
                  w_ih, b_ih.reshape(1, 3 * D), b_hh.reshape(1, 3 * D))


# speedup vs baseline: 1.6589x; 1.6589x over previous
import functools

import jax
import jax.numpy as jnp
from jax import lax
from jax.experimental import pallas as pl
from jax.experimental.pallas import tpu as pltpu
from jax.experimental.pallas import tpu_sc as plsc

N = 10000
E = 320000
D = 128
NC = 2
NS = 16
NW = NC * NS
NPAD = 10240
NPB = NPAD // 128
ROWS_PER_TILE = NPB // NS
EB = 2500
EBP = 2560
BPT = EBP // NW
NBUF = 2
EB2 = 1250
EBP2 = 1280
BPT2 = EBP2 // NW
DEG_CHUNK = 2000

_mesh = plsc.VectorSubcoreMesh(core_axis_name="c", subcore_axis_name="s",
                               num_cores=NC, num_subcores=NS)


def _deg_body(src_hbm, dst_hbm, osrc_hbm, odst_hbm, idxv, hist, vbuf, res,
              shared):
    _Z16F = jnp.zeros((16,), jnp.float32)
    _O16F = jnp.ones((16,), jnp.float32)
    c = lax.axis_index("c")
    s = lax.axis_index("s")

    def _zero_hist(r, _):
        hist[pl.ds(r * 16, 16)] = _Z16F
        return 0
    lax.fori_loop(0, NPAD // 16, _zero_hist, 0)

    per_tile = E // NS

    def _accum():
        def _inner(j, _):
            iv = idxv[pl.ds(j * 16, 16)]
            plsc.addupdate_scatter(hist, [iv], _O16F)
            return 0
        lax.fori_loop(0, DEG_CHUNK // 16, _inner, 0)

    def _chunk(k, _):
        base = s * per_tile + k * DEG_CHUNK

        @pl.when(c == 0)
        def _():
            pltpu.sync_copy(src_hbm.at[pl.ds(base, DEG_CHUNK)], idxv)

        @pl.when(c == 1)
        def _():
            pltpu.sync_copy(dst_hbm.at[pl.ds(base, DEG_CHUNK)], idxv)

        _accum()
        return 0

    lax.fori_loop(0, per_tile // DEG_CHUNK, _chunk, 0)

    pltpu.sync_copy(hist, shared.at[pl.ds(s * NPAD, NPAD)])
    plsc.subcore_barrier()
    seg = NPAD // NS
    for k in range(NS):
        pltpu.sync_copy(shared.at[pl.ds(k * NPAD + s * seg, seg)],
                        vbuf.at[pl.ds(k * seg, seg)])

    def _red(i, _):
        a = vbuf[pl.ds(i * 16, 16)]
        for k in range(1, NS):
            a = a + vbuf[pl.ds(k * seg + i * 16, 16)]
        res[pl.ds(i * 16, 16)] = a
        return 0
    lax.fori_loop(0, seg // 16, _red, 0)

    @pl.when(c == 0)
    def _():
        pltpu.sync_copy(res, osrc_hbm.at[pl.ds(s * seg, seg)])

    @pl.when(c == 1)
    def _():
        pltpu.sync_copy(res, odst_hbm.at[pl.ds(s * seg, seg)])


_deg_call = pl.kernel(
    _deg_body,
    out_type=(jax.ShapeDtypeStruct((NPAD,), jnp.float32),
              jax.ShapeDtypeStruct((NPAD,), jnp.float32)),
    mesh=_mesh,
    compiler_params=pltpu.CompilerParams(needs_layout_passes=False),
    scratch_types=[
        pltpu.VMEM((DEG_CHUNK,), jnp.int32),
        pltpu.VMEM((NPAD,), jnp.float32),
        pltpu.VMEM((NPAD,), jnp.float32),
        pltpu.VMEM((NPAD // NS,), jnp.float32),
        pltpu.VMEM_SHARED((NS * NPAD,), jnp.float32),
    ],
)


def _gspa_body(x_hbm, src2d, dst2d, out_hbm, sidx, didx, rows, acc, sem):
    _Z16F = jnp.zeros((16,), jnp.float32)
    c = lax.axis_index("c")
    s = lax.axis_index("s")
    wid = s * NC + c

    def _zero_rows(r, _):
        for cc in range(8):
            rows[r, pl.ds(cc * 16, 16)] = _Z16F
        return 0
    lax.fori_loop(0, 128, _zero_rows, 0)
    for t in range(NPAD // NS // 128):
        pltpu.sync_copy(rows, acc.at[pl.ds(s * (NPAD // NS) + t * 128, 128)])
    plsc.subcore_barrier()

    def _batch(j, _):
        row = wid + j * NW

        @pl.when(row < EB)
        def _():
            pltpu.sync_copy(src2d.at[row], sidx)
            pltpu.sync_copy(dst2d.at[row], didx.at[0])
            pltpu.async_copy(x_hbm.at[sidx], rows, sem).wait()
            pltpu.sync_copy(rows, acc.at[didx.at[0]], add=True)
        return 0

    lax.fori_loop(0, (EB + NW - 1) // NW, _batch, 0)
    plsc.subcore_barrier()

    sl = pl.ds(s * (NPAD // NS), NPAD // NS)

    @pl.when(c == 0)
    def _():
        pltpu.sync_copy(acc.at[sl], out_hbm.at[0, sl])

    @pl.when(c == 1)
    def _():
        pltpu.sync_copy(acc.at[sl], out_hbm.at[1, sl])


_gspa_call = pl.kernel(
    _gspa_body,
    out_type=jax.ShapeDtypeStruct((2, NPAD, 128), jnp.float32),
    mesh=_mesh,
    scratch_types=[
        pltpu.VMEM((128,), jnp.int32),
        pltpu.VMEM((1, 128), jnp.int32),
        pltpu.VMEM((128, 128), jnp.float32),
        pltpu.VMEM_SHARED((NPAD, 128), jnp.float32),
        pltpu.SemaphoreType.DMA,
    ],
)


def _scale_kernel(feat_ref, deg_ref, o_ref):
    norm = lax.rsqrt(jnp.maximum(deg_ref[...], 1.0))
    o_ref[...] = feat_ref[...] * norm


def _scale(feat, deg_out):
    bn = 1000
    return pl.pallas_call(
        _scale_kernel,
        out_shape=jax.ShapeDtypeStruct((N, D), jnp.float32),
        grid=(N // bn,),
        in_specs=[
            pl.BlockSpec((bn, D), lambda i: (i, 0)),
            pl.BlockSpec((bn, 1), lambda i: (i, 0)),
        ],
        out_specs=pl.BlockSpec((bn, D), lambda i: (i, 0)),
    )(feat, deg_out)


def _dense_kernel(p_ref, deg_ref, w_ref, b_ref, wih_ref, bih_ref, bhh_ref,
                  o_ref):
    norm = lax.rsqrt(jnp.maximum(deg_ref[...], 1.0))
    rst = (p_ref[0] + p_ref[1]) * norm
    h = jnp.dot(rst, w_ref[...], preferred_element_type=jnp.float32) + b_ref[...]
    gi = lax.dot_general(h, wih_ref[...], (((1,), (1,)), ((), ())),
                         preferred_element_type=jnp.float32) + bih_ref[...]
    bhh = bhh_ref[...]
    r = jax.nn.sigmoid(gi[:, 0:D] + bhh[:, 0:D])
    z = jax.nn.sigmoid(gi[:, D:2 * D] + bhh[:, D:2 * D])
    n = jnp.tanh(gi[:, 2 * D:3 * D] + r * bhh[:, 2 * D:3 * D])
    o_ref[...] = jnp.maximum((1.0 - z) * n, 0.0)


def _dense(pacc, deg_in, W, b2, w_ih, bih2, bhh2):
    bn = 400
    return pl.pallas_call(
        _dense_kernel,
        out_shape=jax.ShapeDtypeStruct((N, D), jnp.float32),
        grid=(N // bn,),
        in_specs=[
            pl.BlockSpec((2, bn, D), lambda i: (0, i, 0)),
            pl.BlockSpec((bn, 1), lambda i: (i, 0)),
            pl.BlockSpec((D, D), lambda i: (0, 0)),
            pl.BlockSpec((1, D), lambda i: (0, 0)),
            pl.BlockSpec((3 * D, D), lambda i: (0, 0)),
            pl.BlockSpec((1, 3 * D), lambda i: (0, 0)),
            pl.BlockSpec((1, 3 * D), lambda i: (0, 0)),
        ],
        out_specs=pl.BlockSpec((bn, D), lambda i: (i, 0)),
    )(pacc, deg_in, W, b2, w_ih, bih2, bhh2)


def kernel(feat, edge_index, W, b, w_ih, w_hh, b_ih, b_hh):
    src = edge_index[0]
    dst = edge_index[1]
    dsrc, ddst = _deg_call(src, dst)
    deg_out = dsrc[:N].reshape(N, 1)
    deg_in = ddst[:N].reshape(N, 1)
    x = _scale(feat, deg_out)
    pacc = _gspa_call(x, src.reshape(EB, 128), dst.reshape(EB, 128))
    return _dense(pacc[:, :N, :], deg_in, W, b.reshape(1, D),
                  w_ih, b_ih.reshape(1, 3 * D), b_hh.reshape(1, 3 * D))


# speedup vs baseline: 1.7013x; 1.0256x over previous
import functools

import jax
import jax.numpy as jnp
from jax import lax
from jax.experimental import pallas as pl
from jax.experimental.pallas import tpu as pltpu
from jax.experimental.pallas import tpu_sc as plsc

N = 10000
E = 320000
D = 128
NC = 2
NS = 16
NW = NC * NS
NPAD = 10240
NPB = NPAD // 128
ROWS_PER_TILE = NPB // NS
EB = 2500
EBP = 2560
BPT = EBP // NW
NBUF = 2
EB2 = 1250
EBP2 = 1280
BPT2 = EBP2 // NW
DEG_CHUNK = 2000

_mesh = plsc.VectorSubcoreMesh(core_axis_name="c", subcore_axis_name="s",
                               num_cores=NC, num_subcores=NS)


def _deg_body(src_hbm, dst_hbm, osrc_hbm, odst_hbm, idxv, hist, vbuf, res,
              shared):
    _Z16F = jnp.zeros((16,), jnp.float32)
    _O16F = jnp.ones((16,), jnp.float32)
    c = lax.axis_index("c")
    s = lax.axis_index("s")

    def _zero_hist(r, _):
        hist[pl.ds(r * 16, 16)] = _Z16F
        return 0
    lax.fori_loop(0, NPAD // 16, _zero_hist, 0)

    per_tile = E // NS

    def _accum():
        def _inner(j, _):
            iv = idxv[pl.ds(j * 16, 16)]
            plsc.addupdate_scatter(hist, [iv], _O16F)
            return 0
        lax.fori_loop(0, DEG_CHUNK // 16, _inner, 0)

    def _chunk(k, _):
        base = s * per_tile + k * DEG_CHUNK

        @pl.when(c == 0)
        def _():
            pltpu.sync_copy(src_hbm.at[pl.ds(base, DEG_CHUNK)], idxv)

        @pl.when(c == 1)
        def _():
            pltpu.sync_copy(dst_hbm.at[pl.ds(base, DEG_CHUNK)], idxv)

        _accum()
        return 0

    lax.fori_loop(0, per_tile // DEG_CHUNK, _chunk, 0)

    pltpu.sync_copy(hist, shared.at[pl.ds(s * NPAD, NPAD)])
    plsc.subcore_barrier()
    seg = NPAD // NS
    for k in range(NS):
        pltpu.sync_copy(shared.at[pl.ds(k * NPAD + s * seg, seg)],
                        vbuf.at[pl.ds(k * seg, seg)])

    def _red(i, _):
        a = vbuf[pl.ds(i * 16, 16)]
        for k in range(1, NS):
            a = a + vbuf[pl.ds(k * seg + i * 16, 16)]
        res[pl.ds(i * 16, 16)] = a
        return 0
    lax.fori_loop(0, seg // 16, _red, 0)

    @pl.when(c == 0)
    def _():
        pltpu.sync_copy(res, osrc_hbm.at[pl.ds(s * seg, seg)])

    @pl.when(c == 1)
    def _():
        pltpu.sync_copy(res, odst_hbm.at[pl.ds(s * seg, seg)])


_deg_call = pl.kernel(
    _deg_body,
    out_type=(jax.ShapeDtypeStruct((NPAD,), jnp.float32),
              jax.ShapeDtypeStruct((NPAD,), jnp.float32)),
    mesh=_mesh,
    compiler_params=pltpu.CompilerParams(needs_layout_passes=False),
    scratch_types=[
        pltpu.VMEM((DEG_CHUNK,), jnp.int32),
        pltpu.VMEM((NPAD,), jnp.float32),
        pltpu.VMEM((NPAD,), jnp.float32),
        pltpu.VMEM((NPAD // NS,), jnp.float32),
        pltpu.VMEM_SHARED((NS * NPAD,), jnp.float32),
    ],
)


def _gspa_body(x_hbm, src_hbm, dst_hbm, out_hbm, sidx, didx, rows, acc, sem):
    _Z16F = jnp.zeros((16,), jnp.float32)
    c = lax.axis_index("c")
    s = lax.axis_index("s")
    wid = s * NC + c

    def _zero_rows(r, _):
        for cc in range(8):
            rows[r, pl.ds(cc * 16, 16)] = _Z16F
        return 0
    lax.fori_loop(0, 128, _zero_rows, 0)
    for t in range(NPAD // NS // 128):
        pltpu.sync_copy(rows, acc.at[pl.ds(s * (NPAD // NS) + t * 128, 128)])
    plsc.subcore_barrier()

    def _batch(j, _):
        row = wid + j * NW

        @pl.when(row < EB)
        def _():
            pltpu.sync_copy(src_hbm.at[pl.ds(row * 128, 128)], sidx)
            pltpu.sync_copy(dst_hbm.at[pl.ds(row * 128, 128)], didx.at[0])
            pltpu.async_copy(x_hbm.at[sidx], rows, sem).wait()
            pltpu.sync_copy(rows, acc.at[didx.at[0]], add=True)
        return 0

    lax.fori_loop(0, (EB + NW - 1) // NW, _batch, 0)
    plsc.subcore_barrier()

    sl = pl.ds(s * (NPAD // NS), NPAD // NS)

    @pl.when(c == 0)
    def _():
        pltpu.sync_copy(acc.at[sl], out_hbm.at[0, sl])

    @pl.when(c == 1)
    def _():
        pltpu.sync_copy(acc.at[sl], out_hbm.at[1, sl])


_gspa_call = pl.kernel(
    _gspa_body,
    out_type=jax.ShapeDtypeStruct((2, NPAD, 128), jnp.float32),
    mesh=_mesh,
    scratch_types=[
        pltpu.VMEM((128,), jnp.int32),
        pltpu.VMEM((1, 128), jnp.int32),
        pltpu.VMEM((128, 128), jnp.float32),
        pltpu.VMEM_SHARED((NPAD, 128), jnp.float32),
        pltpu.SemaphoreType.DMA,
    ],
)


def _scale_kernel(feat_ref, deg_ref, o_ref):
    norm = lax.rsqrt(jnp.maximum(deg_ref[...], 1.0))
    o_ref[...] = feat_ref[...] * norm


def _scale(feat, deg_out):
    bn = 1000
    return pl.pallas_call(
        _scale_kernel,
        out_shape=jax.ShapeDtypeStruct((N, D), jnp.float32),
        grid=(N // bn,),
        in_specs=[
            pl.BlockSpec((bn, D), lambda i: (i, 0)),
            pl.BlockSpec((bn, 1), lambda i: (i, 0)),
        ],
        out_specs=pl.BlockSpec((bn, D), lambda i: (i, 0)),
    )(feat, deg_out)


def _dense_kernel(p_ref, deg_ref, w_ref, b_ref, wih_ref, bih_ref, bhh_ref,
                  o_ref):
    norm = lax.rsqrt(jnp.maximum(deg_ref[...], 1.0))
    rst = (p_ref[0] + p_ref[1]) * norm
    h = jnp.dot(rst, w_ref[...], preferred_element_type=jnp.float32) + b_ref[...]
    gi = lax.dot_general(h, wih_ref[...], (((1,), (1,)), ((), ())),
                         preferred_element_type=jnp.float32) + bih_ref[...]
    bhh = bhh_ref[...]
    r = jax.nn.sigmoid(gi[:, 0:D] + bhh[:, 0:D])
    z = jax.nn.sigmoid(gi[:, D:2 * D] + bhh[:, D:2 * D])
    n = jnp.tanh(gi[:, 2 * D:3 * D] + r * bhh[:, 2 * D:3 * D])
    o_ref[...] = jnp.maximum((1.0 - z) * n, 0.0)


def _dense(pacc, deg_in, W, b2, w_ih, bih2, bhh2):
    bn = 400
    return pl.pallas_call(
        _dense_kernel,
        out_shape=jax.ShapeDtypeStruct((N, D), jnp.float32),
        grid=(N // bn,),
        in_specs=[
            pl.BlockSpec((2, bn, D), lambda i: (0, i, 0)),
            pl.BlockSpec((bn, 1), lambda i: (i, 0)),
            pl.BlockSpec((D, D), lambda i: (0, 0)),
            pl.BlockSpec((1, D), lambda i: (0, 0)),
            pl.BlockSpec((3 * D, D), lambda i: (0, 0)),
            pl.BlockSpec((1, 3 * D), lambda i: (0, 0)),
            pl.BlockSpec((1, 3 * D), lambda i: (0, 0)),
        ],
        out_specs=pl.BlockSpec((bn, D), lambda i: (i, 0)),
    )(pacc, deg_in, W, b2, w_ih, bih2, bhh2)


def kernel(feat, edge_index, W, b, w_ih, w_hh, b_ih, b_hh):
    src = edge_index[0]
    dst = edge_index[1]
    dsrc, ddst = _deg_call(src, dst)
    x = _scale(feat, dsrc.reshape(NPAD, 1))
    pacc = _gspa_call(x, src, dst)
    return _dense(pacc, ddst.reshape(NPAD, 1), W, b.reshape(1, D),
                  w_ih, b_ih.reshape(1, 3 * D), b_hh.reshape(1, 3 * D))


# speedup vs baseline: 1.9366x; 1.1383x over previous
import functools

import jax
import jax.numpy as jnp
from jax import lax
from jax.experimental import pallas as pl
from jax.experimental.pallas import tpu as pltpu
from jax.experimental.pallas import tpu_sc as plsc

N = 10000
E = 320000
D = 128
NC = 2
NS = 16
NW = NC * NS
NPAD = 10240
NPB = NPAD // 128
ROWS_PER_TILE = NPB // NS
EB = 2500
EBP = 2560
BPT = EBP // NW
NBUF = 2
EB2 = 1250
EBP2 = 1280
BPT2 = EBP2 // NW
DEG_CHUNK = 2000

_mesh = plsc.VectorSubcoreMesh(core_axis_name="c", subcore_axis_name="s",
                               num_cores=NC, num_subcores=NS)


def _deg_body(src_hbm, dst_hbm, osrc_hbm, odst_hbm, idxv, hist, vbuf, res,
              shared):
    _Z16F = jnp.zeros((16,), jnp.float32)
    _O16F = jnp.ones((16,), jnp.float32)
    c = lax.axis_index("c")
    s = lax.axis_index("s")

    def _zero_hist(r, _):
        hist[pl.ds(r * 16, 16)] = _Z16F
        return 0
    lax.fori_loop(0, NPAD // 16, _zero_hist, 0)

    per_tile = E // NS

    def _accum():
        def _inner(j, _):
            iv = idxv[pl.ds(j * 16, 16)]
            plsc.addupdate_scatter(hist, [iv], _O16F)
            return 0
        lax.fori_loop(0, DEG_CHUNK // 16, _inner, 0)

    def _chunk(k, _):
        base = s * per_tile + k * DEG_CHUNK

        @pl.when(c == 0)
        def _():
            pltpu.sync_copy(src_hbm.at[pl.ds(base, DEG_CHUNK)], idxv)

        @pl.when(c == 1)
        def _():
            pltpu.sync_copy(dst_hbm.at[pl.ds(base, DEG_CHUNK)], idxv)

        _accum()
        return 0

    lax.fori_loop(0, per_tile // DEG_CHUNK, _chunk, 0)

    pltpu.sync_copy(hist, shared.at[pl.ds(s * NPAD, NPAD)])
    plsc.subcore_barrier()
    seg = NPAD // NS
    for k in range(NS):
        pltpu.sync_copy(shared.at[pl.ds(k * NPAD + s * seg, seg)],
                        vbuf.at[pl.ds(k * seg, seg)])

    def _red(i, _):
        a = vbuf[pl.ds(i * 16, 16)]
        for k in range(1, NS):
            a = a + vbuf[pl.ds(k * seg + i * 16, 16)]
        res[pl.ds(i * 16, 16)] = a
        return 0
    lax.fori_loop(0, seg // 16, _red, 0)

    @pl.when(c == 0)
    def _():
        pltpu.sync_copy(res, osrc_hbm.at[pl.ds(s * seg, seg)])

    @pl.when(c == 1)
    def _():
        pltpu.sync_copy(res, odst_hbm.at[pl.ds(s * seg, seg)])


_deg_call = pl.kernel(
    _deg_body,
    out_type=(jax.ShapeDtypeStruct((NPAD,), jnp.float32),
              jax.ShapeDtypeStruct((NPAD,), jnp.float32)),
    mesh=_mesh,
    compiler_params=pltpu.CompilerParams(needs_layout_passes=False),
    scratch_types=[
        pltpu.VMEM((DEG_CHUNK,), jnp.int32),
        pltpu.VMEM((NPAD,), jnp.float32),
        pltpu.VMEM((NPAD,), jnp.float32),
        pltpu.VMEM((NPAD // NS,), jnp.float32),
        pltpu.VMEM_SHARED((NS * NPAD,), jnp.float32),
    ],
)


def _gspa_body(x_hbm, src_hbm, dst_hbm, out_hbm, sidx, didx, rows, acc, sem):
    _Z16F = jnp.zeros((16,), jnp.float32)
    c = lax.axis_index("c")
    s = lax.axis_index("s")
    wid = s * NC + c

    def _zero_rows(r, _):
        for cc in range(8):
            rows[r, pl.ds(cc * 16, 16)] = _Z16F
        return 0
    lax.fori_loop(0, 128, _zero_rows, 0)
    for t in range(NPAD // NS // 128):
        pltpu.sync_copy(rows.at[pl.ds(0, 128)],
                        acc.at[pl.ds(s * (NPAD // NS) + t * 128, 128)])
    plsc.subcore_barrier()

    def _batch(j, _):
        row = wid + j * NW

        @pl.when(row < EB2)
        def _():
            pltpu.sync_copy(src_hbm.at[pl.ds(row * 256, 256)], sidx)
            pltpu.sync_copy(dst_hbm.at[pl.ds(row * 256, 128)], didx.at[0])
            pltpu.sync_copy(dst_hbm.at[pl.ds(row * 256 + 128, 128)],
                            didx.at[1])
            pltpu.async_copy(x_hbm.at[sidx], rows, sem).wait()
            pltpu.sync_copy(rows.at[pl.ds(0, 128)], acc.at[didx.at[0]],
                            add=True)
            pltpu.sync_copy(rows.at[pl.ds(128, 128)], acc.at[didx.at[1]],
                            add=True)
        return 0

    lax.fori_loop(0, (EB2 + NW - 1) // NW, _batch, 0)
    plsc.subcore_barrier()

    sl = pl.ds(s * (NPAD // NS), NPAD // NS)

    @pl.when(c == 0)
    def _():
        pltpu.sync_copy(acc.at[sl], out_hbm.at[0, sl])

    @pl.when(c == 1)
    def _():
        pltpu.sync_copy(acc.at[sl], out_hbm.at[1, sl])


_gspa_call = pl.kernel(
    _gspa_body,
    out_type=jax.ShapeDtypeStruct((2, NPAD, 128), jnp.float32),
    mesh=_mesh,
    scratch_types=[
        pltpu.VMEM((256,), jnp.int32),
        pltpu.VMEM((2, 128), jnp.int32),
        pltpu.VMEM((256, 128), jnp.float32),
        pltpu.VMEM_SHARED((NPAD, 128), jnp.float32),
        pltpu.SemaphoreType.DMA,
    ],
)


def _scale_kernel(feat_ref, deg_ref, o_ref):
    norm = lax.rsqrt(jnp.maximum(deg_ref[...], 1.0))
    o_ref[...] = feat_ref[...] * norm


def _scale(feat, deg_out):
    bn = 1000
    return pl.pallas_call(
        _scale_kernel,
        out_shape=jax.ShapeDtypeStruct((N, D), jnp.float32),
        grid=(N // bn,),
        in_specs=[
            pl.BlockSpec((bn, D), lambda i: (i, 0)),
            pl.BlockSpec((bn, 1), lambda i: (i, 0)),
        ],
        out_specs=pl.BlockSpec((bn, D), lambda i: (i, 0)),
    )(feat, deg_out)


def _dense_kernel(p_ref, deg_ref, w_ref, b_ref, wih_ref, bih_ref, bhh_ref,
                  o_ref):
    norm = lax.rsqrt(jnp.maximum(deg_ref[...], 1.0))
    rst = (p_ref[0] + p_ref[1]) * norm
    h = jnp.dot(rst, w_ref[...], preferred_element_type=jnp.float32) + b_ref[...]
    gi = lax.dot_general(h, wih_ref[...], (((1,), (1,)), ((), ())),
                         preferred_element_type=jnp.float32) + bih_ref[...]
    bhh = bhh_ref[...]
    r = jax.nn.sigmoid(gi[:, 0:D] + bhh[:, 0:D])
    z = jax.nn.sigmoid(gi[:, D:2 * D] + bhh[:, D:2 * D])
    n = jnp.tanh(gi[:, 2 * D:3 * D] + r * bhh[:, 2 * D:3 * D])
    o_ref[...] = jnp.maximum((1.0 - z) * n, 0.0)


def _dense(pacc, deg_in, W, b2, w_ih, bih2, bhh2):
    bn = 400
    return pl.pallas_call(
        _dense_kernel,
        out_shape=jax.ShapeDtypeStruct((N, D), jnp.float32),
        grid=(N // bn,),
        in_specs=[
            pl.BlockSpec((2, bn, D), lambda i: (0, i, 0)),
            pl.BlockSpec((bn, 1), lambda i: (i, 0)),
            pl.BlockSpec((D, D), lambda i: (0, 0)),
            pl.BlockSpec((1, D), lambda i: (0, 0)),
            pl.BlockSpec((3 * D, D), lambda i: (0, 0)),
            pl.BlockSpec((1, 3 * D), lambda i: (0, 0)),
            pl.BlockSpec((1, 3 * D), lambda i: (0, 0)),
        ],
        out_specs=pl.BlockSpec((bn, D), lambda i: (i, 0)),
    )(pacc, deg_in, W, b2, w_ih, bih2, bhh2)


def kernel(feat, edge_index, W, b, w_ih, w_hh, b_ih, b_hh):
    src = edge_index[0]
    dst = edge_index[1]
    dsrc, ddst = _deg_call(src, dst)
    x = _scale(feat, dsrc.reshape(NPAD, 1))
    pacc = _gspa_call(x, src, dst)
    return _dense(pacc, ddst.reshape(NPAD, 1), W, b.reshape(1, D),
                  w_ih, b_ih.reshape(1, 3 * D), b_hh.reshape(1, 3 * D))


# speedup vs baseline: 2.1248x; 1.0972x over previous
import functools

import jax
import jax.numpy as jnp
from jax import lax
from jax.experimental import pallas as pl
from jax.experimental.pallas import tpu as pltpu
from jax.experimental.pallas import tpu_sc as plsc

N = 10000
E = 320000
D = 128
NC = 2
NS = 16
NW = NC * NS
NPAD = 10240
NPB = NPAD // 128
ROWS_PER_TILE = NPB // NS
EB = 2500
EBP = 2560
BPT = EBP // NW
NBUF = 2
EB2 = 1250
EBP2 = 1280
BPT2 = EBP2 // NW
DEG_CHUNK = 2000

_mesh = plsc.VectorSubcoreMesh(core_axis_name="c", subcore_axis_name="s",
                               num_cores=NC, num_subcores=NS)


def _deg_body(src_hbm, dst_hbm, os0_hbm, os1_hbm, od0_hbm, od1_hbm,
              idxv, hist_s, hist_d, vbuf, res, shared_s, shared_d):
    _Z16F = jnp.zeros((16,), jnp.float32)
    _O16F = jnp.ones((16,), jnp.float32)
    c = lax.axis_index("c")
    s = lax.axis_index("s")

    def _zero(r, _):
        hist_s[pl.ds(r * 16, 16)] = _Z16F
        hist_d[pl.ds(r * 16, 16)] = _Z16F
        return 0
    lax.fori_loop(0, NPAD // 16, _zero, 0)

    half = E // NC
    per_tile = half // NS

    def _accum(hist):
        def _inner(j, _):
            iv = idxv[pl.ds(j * 16, 16)]
            plsc.addupdate_scatter(hist, [iv], _O16F)
            return 0
        lax.fori_loop(0, DEG_CHUNK // 16, _inner, 0)

    def _chunk(k, _):
        base = c * half + s * per_tile + k * DEG_CHUNK
        pltpu.sync_copy(src_hbm.at[pl.ds(base, DEG_CHUNK)], idxv)
        _accum(hist_s)
        pltpu.sync_copy(dst_hbm.at[pl.ds(base, DEG_CHUNK)], idxv)
        _accum(hist_d)
        return 0

    lax.fori_loop(0, per_tile // DEG_CHUNK, _chunk, 0)

    pltpu.sync_copy(hist_s, shared_s.at[pl.ds(s * NPAD, NPAD)])
    pltpu.sync_copy(hist_d, shared_d.at[pl.ds(s * NPAD, NPAD)])
    plsc.subcore_barrier()
    seg = NPAD // NS

    def _reduce_to(shared, out_ref):
        for k in range(NS):
            pltpu.sync_copy(shared.at[pl.ds(k * NPAD + s * seg, seg)],
                            vbuf.at[pl.ds(k * seg, seg)])

        def _red(i, _):
            a = vbuf[pl.ds(i * 16, 16)]
            for k in range(1, NS):
                a = a + vbuf[pl.ds(k * seg + i * 16, 16)]
            res[pl.ds(i * 16, 16)] = a
            return 0
        lax.fori_loop(0, seg // 16, _red, 0)
        pltpu.sync_copy(res, out_ref.at[pl.ds(s * seg, seg)])

    @pl.when(c == 0)
    def _():
        _reduce_to(shared_s, os0_hbm)
        _reduce_to(shared_d, od0_hbm)

    @pl.when(c == 1)
    def _():
        _reduce_to(shared_s, os1_hbm)
        _reduce_to(shared_d, od1_hbm)


_deg_call = pl.kernel(
    _deg_body,
    out_type=(jax.ShapeDtypeStruct((NPAD,), jnp.float32),
              jax.ShapeDtypeStruct((NPAD,), jnp.float32),
              jax.ShapeDtypeStruct((NPAD,), jnp.float32),
              jax.ShapeDtypeStruct((NPAD,), jnp.float32)),
    mesh=_mesh,
    compiler_params=pltpu.CompilerParams(needs_layout_passes=False),
    scratch_types=[
        pltpu.VMEM((DEG_CHUNK,), jnp.int32),
        pltpu.VMEM((NPAD,), jnp.float32),
        pltpu.VMEM((NPAD,), jnp.float32),
        pltpu.VMEM((NPAD,), jnp.float32),
        pltpu.VMEM((NPAD // NS,), jnp.float32),
        pltpu.VMEM_SHARED((NS * NPAD,), jnp.float32),
        pltpu.VMEM_SHARED((NS * NPAD,), jnp.float32),
    ],
)


def _gspa_body(x_hbm, src_hbm, dst_hbm, out_hbm, sidx, didx, rows, acc, sem):
    _Z16F = jnp.zeros((16,), jnp.float32)
    c = lax.axis_index("c")
    s = lax.axis_index("s")
    wid = s * NC + c

    def _zero_rows(r, _):
        for cc in range(8):
            rows[r, pl.ds(cc * 16, 16)] = _Z16F
        return 0
    lax.fori_loop(0, 128, _zero_rows, 0)
    for t in range(NPAD // NS // 128):
        pltpu.sync_copy(rows.at[pl.ds(0, 128)],
                        acc.at[pl.ds(s * (NPAD // NS) + t * 128, 128)])
    plsc.subcore_barrier()

    def _batch(j, _):
        row = wid + j * NW

        @pl.when(row < EB2)
        def _():
            pltpu.sync_copy(src_hbm.at[pl.ds(row * 256, 256)], sidx)
            d = pltpu.async_copy(x_hbm.at[sidx], rows, sem)
            pltpu.sync_copy(dst_hbm.at[pl.ds(row * 256, 128)], didx.at[0])
            pltpu.sync_copy(dst_hbm.at[pl.ds(row * 256 + 128, 128)],
                            didx.at[1])
            d.wait()
            pltpu.sync_copy(rows.at[pl.ds(0, 128)], acc.at[didx.at[0]],
                            add=True)
            pltpu.sync_copy(rows.at[pl.ds(128, 128)], acc.at[didx.at[1]],
                            add=True)
        return 0

    lax.fori_loop(0, (EB2 + NW - 1) // NW, _batch, 0)
    plsc.subcore_barrier()

    sl = pl.ds(s * (NPAD // NS), NPAD // NS)

    @pl.when(c == 0)
    def _():
        pltpu.sync_copy(acc.at[sl], out_hbm.at[0, sl])

    @pl.when(c == 1)
    def _():
        pltpu.sync_copy(acc.at[sl], out_hbm.at[1, sl])


_gspa_call = pl.kernel(
    _gspa_body,
    out_type=jax.ShapeDtypeStruct((2, NPAD, 128), jnp.float32),
    mesh=_mesh,
    scratch_types=[
        pltpu.VMEM((256,), jnp.int32),
        pltpu.VMEM((2, 128), jnp.int32),
        pltpu.VMEM((256, 128), jnp.float32),
        pltpu.VMEM_SHARED((NPAD, 128), jnp.float32),
        pltpu.SemaphoreType.DMA,
    ],
)


def _scale_kernel(feat_ref, d0_ref, d1_ref, o_ref):
    norm = lax.rsqrt(jnp.maximum(d0_ref[...] + d1_ref[...], 1.0))
    o_ref[...] = feat_ref[...] * norm


def _scale(feat, d0, d1):
    bn = 1000
    return pl.pallas_call(
        _scale_kernel,
        out_shape=jax.ShapeDtypeStruct((N, D), jnp.float32),
        grid=(N // bn,),
        in_specs=[
            pl.BlockSpec((bn, D), lambda i: (i, 0)),
            pl.BlockSpec((bn, 1), lambda i: (i, 0)),
            pl.BlockSpec((bn, 1), lambda i: (i, 0)),
        ],
        out_specs=pl.BlockSpec((bn, D), lambda i: (i, 0)),
    )(feat, d0, d1)


def _dense_kernel(p_ref, d0_ref, d1_ref, w_ref, b_ref, wih_ref, bih_ref,
                  bhh_ref, o_ref):
    norm = lax.rsqrt(jnp.maximum(d0_ref[...] + d1_ref[...], 1.0))
    rst = (p_ref[0] + p_ref[1]) * norm
    h = jnp.dot(rst, w_ref[...], preferred_element_type=jnp.float32) + b_ref[...]
    gi = lax.dot_general(h, wih_ref[...], (((1,), (1,)), ((), ())),
                         preferred_element_type=jnp.float32) + bih_ref[...]
    bhh = bhh_ref[...]
    r = jax.nn.sigmoid(gi[:, 0:D] + bhh[:, 0:D])
    z = jax.nn.sigmoid(gi[:, D:2 * D] + bhh[:, D:2 * D])
    n = jnp.tanh(gi[:, 2 * D:3 * D] + r * bhh[:, 2 * D:3 * D])
    o_ref[...] = jnp.maximum((1.0 - z) * n, 0.0)


def _dense(pacc, dd0, dd1, W, b2, w_ih, bih2, bhh2):
    bn = 400
    return pl.pallas_call(
        _dense_kernel,
        out_shape=jax.ShapeDtypeStruct((N, D), jnp.float32),
        grid=(N // bn,),
        in_specs=[
            pl.BlockSpec((2, bn, D), lambda i: (0, i, 0)),
            pl.BlockSpec((bn, 1), lambda i: (i, 0)),
            pl.BlockSpec((bn, 1), lambda i: (i, 0)),
            pl.BlockSpec((D, D), lambda i: (0, 0)),
            pl.BlockSpec((1, D), lambda i: (0, 0)),
            pl.BlockSpec((3 * D, D), lambda i: (0, 0)),
            pl.BlockSpec((1, 3 * D), lambda i: (0, 0)),
            pl.BlockSpec((1, 3 * D), lambda i: (0, 0)),
        ],
        out_specs=pl.BlockSpec((bn, D), lambda i: (i, 0)),
    )(pacc, dd0, dd1, W, b2, w_ih, bih2, bhh2)


def kernel(feat, edge_index, W, b, w_ih, w_hh, b_ih, b_hh):
    src = edge_index[0]
    dst = edge_index[1]
    ds0, ds1, dd0, dd1 = _deg_call(src, dst)
    x = _scale(feat, ds0.reshape(NPAD, 1), ds1.reshape(NPAD, 1))
    pacc = _gspa_call(x, src, dst)
    return _dense(pacc, dd0.reshape(NPAD, 1), dd1.reshape(NPAD, 1), W,
                  b.reshape(1, D), w_ih, b_ih.reshape(1, 3 * D),
                  b_hh.reshape(1, 3 * D))


# speedup vs baseline: 2.2736x; 1.0700x over previous
import functools

import jax
import jax.numpy as jnp
from jax import lax
from jax.experimental import pallas as pl
from jax.experimental.pallas import tpu as pltpu
from jax.experimental.pallas import tpu_sc as plsc

N = 10000
E = 320000
D = 128
NC = 2
NS = 16
NW = NC * NS
NPAD = 10240
NPB = NPAD // 128
ROWS_PER_TILE = NPB // NS
EB = 2500
EBP = 2560
BPT = EBP // NW
NBUF = 2
EB2 = 1250
EBP2 = 1280
BPT2 = EBP2 // NW
DEG_CHUNK = 2000

_mesh = plsc.VectorSubcoreMesh(core_axis_name="c", subcore_axis_name="s",
                               num_cores=NC, num_subcores=NS)


def _deg_body(src_hbm, dst_hbm, os0_hbm, os1_hbm, od0_hbm, od1_hbm,
              idxv, hist_s, hist_d, vbuf, res, shared_s, shared_d):
    _Z16F = jnp.zeros((16,), jnp.float32)
    _O16F = jnp.ones((16,), jnp.float32)
    c = lax.axis_index("c")
    s = lax.axis_index("s")

    def _zero(r, _):
        hist_s[pl.ds(r * 16, 16)] = _Z16F
        hist_d[pl.ds(r * 16, 16)] = _Z16F
        return 0
    lax.fori_loop(0, NPAD // 16, _zero, 0)

    half = E // NC
    per_tile = half // NS

    def _accum(hist):
        def _inner(j, _):
            iv = idxv[pl.ds(j * 16, 16)]
            plsc.addupdate_scatter(hist, [iv], _O16F)
            return 0
        lax.fori_loop(0, DEG_CHUNK // 16, _inner, 0)

    def _chunk(k, _):
        base = c * half + s * per_tile + k * DEG_CHUNK
        pltpu.sync_copy(src_hbm.at[pl.ds(base, DEG_CHUNK)], idxv)
        _accum(hist_s)
        pltpu.sync_copy(dst_hbm.at[pl.ds(base, DEG_CHUNK)], idxv)
        _accum(hist_d)
        return 0

    lax.fori_loop(0, per_tile // DEG_CHUNK, _chunk, 0)

    pltpu.sync_copy(hist_s, shared_s.at[pl.ds(s * NPAD, NPAD)])
    pltpu.sync_copy(hist_d, shared_d.at[pl.ds(s * NPAD, NPAD)])
    plsc.subcore_barrier()
    seg = NPAD // NS

    def _reduce_to(shared, out_ref):
        for k in range(NS):
            pltpu.sync_copy(shared.at[pl.ds(k * NPAD + s * seg, seg)],
                            vbuf.at[pl.ds(k * seg, seg)])

        def _red(i, _):
            a = vbuf[pl.ds(i * 16, 16)]
            for k in range(1, NS):
                a = a + vbuf[pl.ds(k * seg + i * 16, 16)]
            res[pl.ds(i * 16, 16)] = a
            return 0
        lax.fori_loop(0, seg // 16, _red, 0)
        pltpu.sync_copy(res, out_ref.at[pl.ds(s * seg, seg)])

    @pl.when(c == 0)
    def _():
        _reduce_to(shared_s, os0_hbm)
        _reduce_to(shared_d, od0_hbm)

    @pl.when(c == 1)
    def _():
        _reduce_to(shared_s, os1_hbm)
        _reduce_to(shared_d, od1_hbm)


_deg_call = pl.kernel(
    _deg_body,
    out_type=(jax.ShapeDtypeStruct((NPAD,), jnp.float32),
              jax.ShapeDtypeStruct((NPAD,), jnp.float32),
              jax.ShapeDtypeStruct((NPAD,), jnp.float32),
              jax.ShapeDtypeStruct((NPAD,), jnp.float32)),
    mesh=_mesh,
    compiler_params=pltpu.CompilerParams(needs_layout_passes=False),
    scratch_types=[
        pltpu.VMEM((DEG_CHUNK,), jnp.int32),
        pltpu.VMEM((NPAD,), jnp.float32),
        pltpu.VMEM((NPAD,), jnp.float32),
        pltpu.VMEM((NPAD,), jnp.float32),
        pltpu.VMEM((NPAD // NS,), jnp.float32),
        pltpu.VMEM_SHARED((NS * NPAD,), jnp.float32),
        pltpu.VMEM_SHARED((NS * NPAD,), jnp.float32),
    ],
)


def _gspa_body(x_hbm, src_hbm, dst_hbm, out_hbm, sidx0, sidx1, didx, rows,
               acc, sem, ssem0, ssem1):
    _Z16F = jnp.zeros((16,), jnp.float32)
    c = lax.axis_index("c")
    s = lax.axis_index("s")
    wid = s * NC + c

    def _zero_rows(r, _):
        for cc in range(8):
            rows[r, pl.ds(cc * 16, 16)] = _Z16F
        return 0
    lax.fori_loop(0, 128, _zero_rows, 0)
    for t in range(NPAD // NS // 128):
        pltpu.sync_copy(rows.at[pl.ds(0, 128)],
                        acc.at[pl.ds(s * (NPAD // NS) + t * 128, 128)])
    plsc.subcore_barrier()

    def _do_batch(j, t2, sidx_t, ssem_t, first):
        row = wid + j * NW

        @pl.when(row < EB2)
        def _():
            if not first:
                pltpu.make_async_copy(src_hbm.at[pl.ds(0, 256)], sidx_t,
                                      ssem_t).wait()
            d = pltpu.async_copy(x_hbm.at[sidx_t], rows, sem)
            pltpu.sync_copy(dst_hbm.at[pl.ds(row * 256, 128)], didx.at[0])
            pltpu.sync_copy(dst_hbm.at[pl.ds(row * 256 + 128, 128)],
                            didx.at[1])
            d.wait()

            @pl.when(row + 2 * NW < EB2)
            def _():
                pltpu.async_copy(
                    src_hbm.at[pl.ds((row + 2 * NW) * 256, 256)],
                    sidx_t, ssem_t)

            pltpu.sync_copy(rows.at[pl.ds(0, 128)], acc.at[didx.at[0]],
                            add=True)
            pltpu.sync_copy(rows.at[pl.ds(128, 128)], acc.at[didx.at[1]],
                            add=True)

    pltpu.sync_copy(src_hbm.at[pl.ds(wid * 256, 256)], sidx0)
    pltpu.async_copy(src_hbm.at[pl.ds((wid + NW) * 256, 256)], sidx1, ssem1)
    _do_batch(0, 0, sidx0, ssem0, True)
    _do_batch(1, 1, sidx1, ssem1, False)

    def _pair(g, _):
        _do_batch(2 * g, 0, sidx0, ssem0, False)
        _do_batch(2 * g + 1, 1, sidx1, ssem1, False)
        return 0

    lax.fori_loop(1, (EB2 + 2 * NW - 1) // (2 * NW), _pair, 0)
    plsc.subcore_barrier()

    sl = pl.ds(s * (NPAD // NS), NPAD // NS)

    @pl.when(c == 0)
    def _():
        pltpu.sync_copy(acc.at[sl], out_hbm.at[0, sl])

    @pl.when(c == 1)
    def _():
        pltpu.sync_copy(acc.at[sl], out_hbm.at[1, sl])


_gspa_call = pl.kernel(
    _gspa_body,
    out_type=jax.ShapeDtypeStruct((2, NPAD, 128), jnp.float32),
    mesh=_mesh,
    scratch_types=[
        pltpu.VMEM((256,), jnp.int32),
        pltpu.VMEM((256,), jnp.int32),
        pltpu.VMEM((2, 128), jnp.int32),
        pltpu.VMEM((256, 128), jnp.float32),
        pltpu.VMEM_SHARED((NPAD, 128), jnp.float32),
        pltpu.SemaphoreType.DMA,
        pltpu.SemaphoreType.DMA,
        pltpu.SemaphoreType.DMA,
    ],
)


def _scale_kernel(feat_ref, d0_ref, d1_ref, o_ref):
    norm = lax.rsqrt(jnp.maximum(d0_ref[...] + d1_ref[...], 1.0))
    o_ref[...] = feat_ref[...] * norm


def _scale(feat, d0, d1):
    bn = 1000
    return pl.pallas_call(
        _scale_kernel,
        out_shape=jax.ShapeDtypeStruct((N, D), jnp.float32),
        grid=(N // bn,),
        in_specs=[
            pl.BlockSpec((bn, D), lambda i: (i, 0)),
            pl.BlockSpec((bn, 1), lambda i: (i, 0)),
            pl.BlockSpec((bn, 1), lambda i: (i, 0)),
        ],
        out_specs=pl.BlockSpec((bn, D), lambda i: (i, 0)),
    )(feat, d0, d1)


def _dense_kernel(p_ref, d0_ref, d1_ref, w_ref, b_ref, wih_ref, bih_ref,
                  bhh_ref, o_ref):
    norm = lax.rsqrt(jnp.maximum(d0_ref[...] + d1_ref[...], 1.0))
    rst = (p_ref[0] + p_ref[1]) * norm
    h = jnp.dot(rst, w_ref[...], preferred_element_type=jnp.float32) + b_ref[...]
    gi = lax.dot_general(h, wih_ref[...], (((1,), (1,)), ((), ())),
                         preferred_element_type=jnp.float32) + bih_ref[...]
    bhh = bhh_ref[...]
    r = jax.nn.sigmoid(gi[:, 0:D] + bhh[:, 0:D])
    z = jax.nn.sigmoid(gi[:, D:2 * D] + bhh[:, D:2 * D])
    n = jnp.tanh(gi[:, 2 * D:3 * D] + r * bhh[:, 2 * D:3 * D])
    o_ref[...] = jnp.maximum((1.0 - z) * n, 0.0)


def _dense(pacc, dd0, dd1, W, b2, w_ih, bih2, bhh2):
    bn = 400
    return pl.pallas_call(
        _dense_kernel,
        out_shape=jax.ShapeDtypeStruct((N, D), jnp.float32),
        grid=(N // bn,),
        in_specs=[
            pl.BlockSpec((2, bn, D), lambda i: (0, i, 0)),
            pl.BlockSpec((bn, 1), lambda i: (i, 0)),
            pl.BlockSpec((bn, 1), lambda i: (i, 0)),
            pl.BlockSpec((D, D), lambda i: (0, 0)),
            pl.BlockSpec((1, D), lambda i: (0, 0)),
            pl.BlockSpec((3 * D, D), lambda i: (0, 0)),
            pl.BlockSpec((1, 3 * D), lambda i: (0, 0)),
            pl.BlockSpec((1, 3 * D), lambda i: (0, 0)),
        ],
        out_specs=pl.BlockSpec((bn, D), lambda i: (i, 0)),
    )(pacc, dd0, dd1, W, b2, w_ih, bih2, bhh2)


def kernel(feat, edge_index, W, b, w_ih, w_hh, b_ih, b_hh):
    src = edge_index[0]
    dst = edge_index[1]
    ds0, ds1, dd0, dd1 = _deg_call(src, dst)
    x = _scale(feat, ds0.reshape(NPAD, 1), ds1.reshape(NPAD, 1))
    pacc = _gspa_call(x, src, dst)
    return _dense(pacc, dd0.reshape(NPAD, 1), dd1.reshape(NPAD, 1), W,
                  b.reshape(1, D), w_ih, b_ih.reshape(1, 3 * D),
                  b_hh.reshape(1, 3 * D))


# speedup vs baseline: 2.2762x; 1.0012x over previous
import jax
import jax.numpy as jnp
from jax import lax
from jax.experimental import pallas as pl
from jax.experimental.pallas import tpu as pltpu
from jax.experimental.pallas import tpu_sc as plsc

N = 10000
E = 320000
D = 128
NC = 2
NS = 16
NW = NC * NS
NPAD = 10240
EB2 = 1250
DEG_CHUNK = 2000

_mesh = plsc.VectorSubcoreMesh(core_axis_name="c", subcore_axis_name="s",
                               num_cores=NC, num_subcores=NS)


def _deg_body(src_hbm, dst_hbm, os0_hbm, os1_hbm, od0_hbm, od1_hbm,
              idxv, hist_s, hist_d, vbuf, res, shared_s, shared_d):
    _Z16F = jnp.zeros((16,), jnp.float32)
    _O16F = jnp.ones((16,), jnp.float32)
    c = lax.axis_index("c")
    s = lax.axis_index("s")

    def _zero(r, _):
        hist_s[pl.ds(r * 16, 16)] = _Z16F
        hist_d[pl.ds(r * 16, 16)] = _Z16F
        return 0
    lax.fori_loop(0, NPAD // 16, _zero, 0)

    half = E // NC
    per_tile = half // NS

    def _accum(hist):
        def _inner(j, _):
            iv = idxv[pl.ds(j * 16, 16)]
            plsc.addupdate_scatter(hist, [iv], _O16F)
            return 0
        lax.fori_loop(0, DEG_CHUNK // 16, _inner, 0)

    def _chunk(k, _):
        base = c * half + s * per_tile + k * DEG_CHUNK
        pltpu.sync_copy(src_hbm.at[pl.ds(base, DEG_CHUNK)], idxv)
        _accum(hist_s)
        pltpu.sync_copy(dst_hbm.at[pl.ds(base, DEG_CHUNK)], idxv)
        _accum(hist_d)
        return 0

    lax.fori_loop(0, per_tile // DEG_CHUNK, _chunk, 0)

    pltpu.sync_copy(hist_s, shared_s.at[pl.ds(s * NPAD, NPAD)])
    pltpu.sync_copy(hist_d, shared_d.at[pl.ds(s * NPAD, NPAD)])
    plsc.subcore_barrier()
    seg = NPAD // NS

    def _reduce_to(shared, out_ref):
        for k in range(NS):
            pltpu.sync_copy(shared.at[pl.ds(k * NPAD + s * seg, seg)],
                            vbuf.at[pl.ds(k * seg, seg)])

        def _red(i, _):
            a = vbuf[pl.ds(i * 16, 16)]
            for k in range(1, NS):
                a = a + vbuf[pl.ds(k * seg + i * 16, 16)]
            res[pl.ds(i * 16, 16)] = a
            return 0
        lax.fori_loop(0, seg // 16, _red, 0)
        pltpu.sync_copy(res, out_ref.at[pl.ds(s * seg, seg)])

    @pl.when(c == 0)
    def _():
        _reduce_to(shared_s, os0_hbm)
        _reduce_to(shared_d, od0_hbm)

    @pl.when(c == 1)
    def _():
        _reduce_to(shared_s, os1_hbm)
        _reduce_to(shared_d, od1_hbm)


_deg_call = pl.kernel(
    _deg_body,
    out_type=(jax.ShapeDtypeStruct((NPAD,), jnp.float32),
              jax.ShapeDtypeStruct((NPAD,), jnp.float32),
              jax.ShapeDtypeStruct((NPAD,), jnp.float32),
              jax.ShapeDtypeStruct((NPAD,), jnp.float32)),
    mesh=_mesh,
    compiler_params=pltpu.CompilerParams(needs_layout_passes=False),
    scratch_types=[
        pltpu.VMEM((DEG_CHUNK,), jnp.int32),
        pltpu.VMEM((NPAD,), jnp.float32),
        pltpu.VMEM((NPAD,), jnp.float32),
        pltpu.VMEM((NPAD,), jnp.float32),
        pltpu.VMEM((NPAD // NS,), jnp.float32),
        pltpu.VMEM_SHARED((NS * NPAD,), jnp.float32),
        pltpu.VMEM_SHARED((NS * NPAD,), jnp.float32),
    ],
)


def _gspa_body(x_hbm, src_hbm, dst_hbm, out_hbm, sidx0, sidx1, didx, rows,
               acc, sem, ssem0, ssem1):
    _Z16F = jnp.zeros((16,), jnp.float32)
    c = lax.axis_index("c")
    s = lax.axis_index("s")
    wid = s * NC + c

    def _zero_rows(r, _):
        for cc in range(8):
            rows[r, pl.ds(cc * 16, 16)] = _Z16F
        return 0
    lax.fori_loop(0, 128, _zero_rows, 0)
    for t in range(NPAD // NS // 128):
        pltpu.sync_copy(rows.at[pl.ds(0, 128)],
                        acc.at[pl.ds(s * (NPAD // NS) + t * 128, 128)])
    plsc.subcore_barrier()

    def _do_batch(j, t2, sidx_t, ssem_t, first):
        row = wid + j * NW

        @pl.when(row < EB2)
        def _():
            if not first:
                pltpu.make_async_copy(src_hbm.at[pl.ds(0, 256)], sidx_t,
                                      ssem_t).wait()
            d = pltpu.async_copy(x_hbm.at[sidx_t], rows, sem)
            pltpu.sync_copy(dst_hbm.at[pl.ds(row * 256, 128)], didx.at[0])
            pltpu.sync_copy(dst_hbm.at[pl.ds(row * 256 + 128, 128)],
                            didx.at[1])
            d.wait()

            @pl.when(row + 2 * NW < EB2)
            def _():
                pltpu.async_copy(
                    src_hbm.at[pl.ds((row + 2 * NW) * 256, 256)],
                    sidx_t, ssem_t)

            pltpu.sync_copy(rows.at[pl.ds(0, 128)], acc.at[didx.at[0]],
                            add=True)
            pltpu.sync_copy(rows.at[pl.ds(128, 128)], acc.at[didx.at[1]],
                            add=True)

    pltpu.sync_copy(src_hbm.at[pl.ds(wid * 256, 256)], sidx0)
    pltpu.async_copy(src_hbm.at[pl.ds((wid + NW) * 256, 256)], sidx1, ssem1)
    _do_batch(0, 0, sidx0, ssem0, True)
    _do_batch(1, 1, sidx1, ssem1, False)

    def _pair(g, _):
        _do_batch(2 * g, 0, sidx0, ssem0, False)
        _do_batch(2 * g + 1, 1, sidx1, ssem1, False)
        return 0

    lax.fori_loop(1, (EB2 + 2 * NW - 1) // (2 * NW), _pair, 0)
    plsc.subcore_barrier()

    sl = pl.ds(s * (NPAD // NS), NPAD // NS)

    @pl.when(c == 0)
    def _():
        pltpu.sync_copy(acc.at[sl], out_hbm.at[0, sl])

    @pl.when(c == 1)
    def _():
        pltpu.sync_copy(acc.at[sl], out_hbm.at[1, sl])


_gspa_call = pl.kernel(
    _gspa_body,
    out_type=jax.ShapeDtypeStruct((2, NPAD, 128), jnp.float32),
    mesh=_mesh,
    scratch_types=[
        pltpu.VMEM((256,), jnp.int32),
        pltpu.VMEM((256,), jnp.int32),
        pltpu.VMEM((2, 128), jnp.int32),
        pltpu.VMEM((256, 128), jnp.float32),
        pltpu.VMEM_SHARED((NPAD, 128), jnp.float32),
        pltpu.SemaphoreType.DMA,
        pltpu.SemaphoreType.DMA,
        pltpu.SemaphoreType.DMA,
    ],
)


def _scale_kernel(feat_ref, d0_ref, d1_ref, o_ref):
    norm = lax.rsqrt(jnp.maximum(d0_ref[...] + d1_ref[...], 1.0))
    o_ref[...] = feat_ref[...] * norm


def _scale(feat, d0, d1):
    bn = 1000
    return pl.pallas_call(
        _scale_kernel,
        out_shape=jax.ShapeDtypeStruct((N, D), jnp.float32),
        grid=(N // bn,),
        in_specs=[
            pl.BlockSpec((bn, D), lambda i: (i, 0)),
            pl.BlockSpec((bn, 1), lambda i: (i, 0)),
            pl.BlockSpec((bn, 1), lambda i: (i, 0)),
        ],
        out_specs=pl.BlockSpec((bn, D), lambda i: (i, 0)),
    )(feat, d0, d1)


def _dense_kernel(p_ref, d0_ref, d1_ref, w_ref, b_ref, wih_ref, bih_ref,
                  bhh_ref, o_ref):
    norm = lax.rsqrt(jnp.maximum(d0_ref[...] + d1_ref[...], 1.0))
    rst = (p_ref[0] + p_ref[1]) * norm
    h = jnp.dot(rst, w_ref[...], preferred_element_type=jnp.float32) + b_ref[...]
    gi = lax.dot_general(h, wih_ref[...], (((1,), (1,)), ((), ())),
                         preferred_element_type=jnp.float32) + bih_ref[...]
    bhh = bhh_ref[...]
    r = jax.nn.sigmoid(gi[:, 0:D] + bhh[:, 0:D])
    z = jax.nn.sigmoid(gi[:, D:2 * D] + bhh[:, D:2 * D])
    n = jnp.tanh(gi[:, 2 * D:3 * D] + r * bhh[:, 2 * D:3 * D])
    o_ref[...] = jnp.maximum((1.0 - z) * n, 0.0)


def _dense(pacc, dd0, dd1, W, b2, w_ih, bih2, bhh2):
    bn = 400
    return pl.pallas_call(
        _dense_kernel,
        out_shape=jax.ShapeDtypeStruct((N, D), jnp.float32),
        grid=(N // bn,),
        in_specs=[
            pl.BlockSpec((2, bn, D), lambda i: (0, i, 0)),
            pl.BlockSpec((bn, 1), lambda i: (i, 0)),
            pl.BlockSpec((bn, 1), lambda i: (i, 0)),
            pl.BlockSpec((D, D), lambda i: (0, 0)),
            pl.BlockSpec((1, D), lambda i: (0, 0)),
            pl.BlockSpec((3 * D, D), lambda i: (0, 0)),
            pl.BlockSpec((1, 3 * D), lambda i: (0, 0)),
            pl.BlockSpec((1, 3 * D), lambda i: (0, 0)),
        ],
        out_specs=pl.BlockSpec((bn, D), lambda i: (i, 0)),
    )(pacc, dd0, dd1, W, b2, w_ih, bih2, bhh2)


def kernel(feat, edge_index, W, b, w_ih, w_hh, b_ih, b_hh):
    src = edge_index[0]
    dst = edge_index[1]
    ds0, ds1, dd0, dd1 = _deg_call(src, dst)
    x = _scale(feat, ds0.reshape(NPAD, 1), ds1.reshape(NPAD, 1))
    pacc = _gspa_call(x, src, dst)
    return _dense(pacc, dd0.reshape(NPAD, 1), dd1.reshape(NPAD, 1), W,
                  b.reshape(1, D), w_ih, b_ih.reshape(1, 3 * D),
                  b_hh.reshape(1, 3 * D))
